# baseline (speedup 1.0000x reference)
import functools

import jax
import jax.numpy as jnp
from jax import lax
from jax.experimental import pallas as pl
from jax.experimental.pallas import tpu as pltpu
from jax.experimental.pallas import tpu_sc as plsc

N_NODES = 10000
N_EDGES = 640000
LANES = 16
NUM_CORES = 2
NUM_SUBCORES = 16
NUM_TILES = NUM_CORES * NUM_SUBCORES
PER_TILE = N_EDGES // NUM_TILES
CHUNK = 4000
N_CHUNKS = PER_TILE // CHUNK
GROUPS = CHUNK // LANES
N_PAD = 10240
SLICE = N_PAD // NUM_SUBCORES


def _lj_kernel_body(x_hbm, y_hbm, z_hbm, idx_i_hbm, idx_j_hbm, species_hbm,
                    ptab_hbm, out_hbm,
                    species_v, energy_v, ptab_v, x_buf0, y_buf0, z_buf0,
                    i_buf0, j_buf0, x_buf1, y_buf1, z_buf1, i_buf1, j_buf1,
                    shared, red_buf, out_buf, sems, csem):
    c = lax.axis_index("c")
    s = lax.axis_index("s")
    wid = c * NUM_SUBCORES + s
    tile_base = wid * PER_TILE

    bufs = [
        (x_buf0, y_buf0, z_buf0, i_buf0, j_buf0),
        (x_buf1, y_buf1, z_buf1, i_buf1, j_buf1),
    ]

    def start_chunk(ci, bi):
        e0 = tile_base + ci * CHUNK
        sem = sems.at[bi]
        xb, yb, zb, ib, jb = bufs[bi]
        return [
            pltpu.async_copy(x_hbm.at[pl.ds(e0, CHUNK)], xb, sem),
            pltpu.async_copy(y_hbm.at[pl.ds(e0, CHUNK)], yb, sem),
            pltpu.async_copy(z_hbm.at[pl.ds(e0, CHUNK)], zb, sem),
            pltpu.async_copy(idx_i_hbm.at[pl.ds(e0, CHUNK)], ib, sem),
            pltpu.async_copy(idx_j_hbm.at[pl.ds(e0, CHUNK)], jb, sem),
        ]

    descs = {0: start_chunk(0, 0), 1: start_chunk(1, 1)}
    spec_copy = pltpu.async_copy(species_hbm, species_v, csem)
    pltpu.sync_copy(ptab_hbm, ptab_v)

    zeros16 = jnp.zeros((LANES,), jnp.float32)

    @plsc.parallel_loop(0, N_PAD // LANES, 1, unroll=8)
    def _(v):
        energy_v[pl.ds(v * LANES, LANES)] = zeros16

    def process_chunk(bi):
        xb, yb, zb, ib, jb = bufs[bi]

        @plsc.parallel_loop(0, GROUPS, 1, unroll=5)
        def _(g):
            base = g * LANES
            i_vec = ib[pl.ds(base, LANES)]
            j_vec = jb[pl.ds(base, LANES)]
            x = xb[pl.ds(base, LANES)]
            y = yb[pl.ds(base, LANES)]
            z = zb[pl.ds(base, LANES)]
            si = plsc.load_gather(species_v, [i_vec])
            sj = plsc.load_gather(species_v, [j_vec])
            p = si * 2 + sj
            s6 = plsc.load_gather(ptab_v, [p])
            fe = plsc.load_gather(ptab_v, [p + 8])
            sh = plsc.load_gather(ptab_v, [p + 16])
            r2 = x * x + y * y + z * z
            r6 = r2 * r2 * r2
            sr6 = s6 / r6
            e_half = (fe * (sr6 * sr6 - sr6) - sh) * 0.5
            plsc.addupdate_scatter(energy_v, [i_vec], e_half)
            plsc.addupdate_scatter(energy_v, [j_vec], e_half)

    spec_copy.wait()
    for ci in range(N_CHUNKS):
        bi = ci % 2
        for d in descs.pop(ci):
            d.wait()
        process_chunk(bi)
        if ci + 2 < N_CHUNKS:
            descs[ci + 2] = start_chunk(ci + 2, bi)

    pltpu.sync_copy(energy_v, shared.at[s])
    plsc.subcore_barrier()
    pltpu.sync_copy(shared.at[:, pl.ds(s * SLICE, SLICE)], red_buf)

    @plsc.parallel_loop(0, SLICE // LANES, 1, unroll=4)
    def _(v):
        col = v * LANES
        acc = red_buf[0, pl.ds(col, LANES)]
        for r in range(1, NUM_SUBCORES):
            acc = acc + red_buf[r, pl.ds(col, LANES)]
        out_buf[pl.ds(col, LANES)] = acc

    pltpu.sync_copy(out_buf, out_hbm.at[c, pl.ds(s * SLICE, SLICE)])


@jax.jit
def _lj_sc(x, y, z, idx_i, idx_j, species, ptab):
    mesh = plsc.VectorSubcoreMesh(core_axis_name="c", subcore_axis_name="s")
    run = pl.kernel(
        _lj_kernel_body,
        out_type=jax.ShapeDtypeStruct((NUM_CORES, N_PAD), jnp.float32),
        mesh=mesh,
        compiler_params=pltpu.CompilerParams(needs_layout_passes=False),
        scratch_types=[
            pltpu.VMEM((N_NODES,), jnp.int32),
            pltpu.VMEM((N_PAD,), jnp.float32),
            pltpu.VMEM((24,), jnp.float32),
            pltpu.VMEM((CHUNK,), jnp.float32),
            pltpu.VMEM((CHUNK,), jnp.float32),
            pltpu.VMEM((CHUNK,), jnp.float32),
            pltpu.VMEM((CHUNK,), jnp.int32),
            pltpu.VMEM((CHUNK,), jnp.int32),
            pltpu.VMEM((CHUNK,), jnp.float32),
            pltpu.VMEM((CHUNK,), jnp.float32),
            pltpu.VMEM((CHUNK,), jnp.float32),
            pltpu.VMEM((CHUNK,), jnp.int32),
            pltpu.VMEM((CHUNK,), jnp.int32),
            pltpu.VMEM_SHARED((NUM_SUBCORES, N_PAD), jnp.float32),
            pltpu.VMEM((NUM_SUBCORES, SLICE), jnp.float32),
            pltpu.VMEM((SLICE,), jnp.float32),
            pltpu.SemaphoreType.DMA((2,)),
            pltpu.SemaphoreType.DMA,
        ],
    )
    return run(x, y, z, idx_i, idx_j, species, ptab)


def kernel(edge_vectors, edge_index, species, lj_table):
    sigma = lj_table[..., 0].reshape(-1)
    eps = lj_table[..., 1].reshape(-1)
    shift = lj_table[..., 2].reshape(-1)
    s3 = sigma * sigma * sigma
    ptab = jnp.zeros((24,), jnp.float32)
    ptab = ptab.at[0:4].set(s3 * s3)
    ptab = ptab.at[8:12].set(4.0 * eps)
    ptab = ptab.at[16:20].set(shift)

    idx = edge_index.astype(jnp.int32)
    ev = edge_vectors.astype(jnp.float32)
    out = _lj_sc(ev[:, 0], ev[:, 1], ev[:, 2],
                 idx[0], idx[1],
                 species.astype(jnp.int32), ptab)
    energy = out[0, :N_NODES] + out[1, :N_NODES]
    return energy.reshape(-1, 1)


# speedup vs baseline: 1.0320x; 1.0320x over previous
import functools

import jax
import jax.numpy as jnp
from jax import lax
from jax.experimental import pallas as pl
from jax.experimental.pallas import tpu as pltpu
from jax.experimental.pallas import tpu_sc as plsc

N_NODES = 10000
N_EDGES = 640000
LANES = 16
NUM_CORES = 2
NUM_SUBCORES = 16
NUM_TILES = NUM_CORES * NUM_SUBCORES
PER_TILE = N_EDGES // NUM_TILES
CHUNK = 2000
N_CHUNKS = PER_TILE // CHUNK
GROUPS = CHUNK // LANES
N_PAD = 10240
SLICE = N_PAD // NUM_SUBCORES


def _lj_kernel_body(x_hbm, y_hbm, z_hbm, idx_i_hbm, idx_j_hbm, species_hbm,
                    ptab_hbm, out_hbm,
                    species_v, energy_v, ptab_v, x_buf0, y_buf0, z_buf0,
                    i_buf0, j_buf0, x_buf1, y_buf1, z_buf1, i_buf1, j_buf1,
                    shared, red_buf, out_buf, sems, csem):
    c = lax.axis_index("c")
    s = lax.axis_index("s")
    wid = c * NUM_SUBCORES + s
    tile_base = wid * PER_TILE

    bufs = [
        (x_buf0, y_buf0, z_buf0, i_buf0, j_buf0),
        (x_buf1, y_buf1, z_buf1, i_buf1, j_buf1),
    ]

    def start_chunk(ci, bi):
        e0 = tile_base + ci * CHUNK
        sem = sems.at[bi]
        xb, yb, zb, ib, jb = bufs[bi]
        return [
            pltpu.async_copy(x_hbm.at[pl.ds(e0, CHUNK)], xb, sem),
            pltpu.async_copy(y_hbm.at[pl.ds(e0, CHUNK)], yb, sem),
            pltpu.async_copy(z_hbm.at[pl.ds(e0, CHUNK)], zb, sem),
            pltpu.async_copy(idx_i_hbm.at[pl.ds(e0, CHUNK)], ib, sem),
            pltpu.async_copy(idx_j_hbm.at[pl.ds(e0, CHUNK)], jb, sem),
        ]

    descs = {0: start_chunk(0, 0), 1: start_chunk(1, 1)}
    spec_copy = pltpu.async_copy(species_hbm, species_v, csem)
    pltpu.sync_copy(ptab_hbm, ptab_v)

    def splat(k):
        return plsc.load_gather(ptab_v, [jnp.full((LANES,), k, jnp.int32)])

    s6_00, s6_01, s6_10, s6_11 = splat(0), splat(1), splat(2), splat(3)
    fe_00, fe_01, fe_10, fe_11 = splat(8), splat(9), splat(10), splat(11)
    sh_00, sh_01, sh_10, sh_11 = splat(16), splat(17), splat(18), splat(19)

    zeros16 = jnp.zeros((LANES,), jnp.float32)

    @plsc.parallel_loop(0, N_PAD // LANES, 1, unroll=8)
    def _(v):
        energy_v[pl.ds(v * LANES, LANES)] = zeros16

    def process_chunk(bi):
        xb, yb, zb, ib, jb = bufs[bi]

        @plsc.parallel_loop(0, GROUPS, 1, unroll=5)
        def _(g):
            base = g * LANES
            i_vec = ib[pl.ds(base, LANES)]
            j_vec = jb[pl.ds(base, LANES)]
            x = xb[pl.ds(base, LANES)]
            y = yb[pl.ds(base, LANES)]
            z = zb[pl.ds(base, LANES)]
            si = plsc.load_gather(species_v, [i_vec])
            sj = plsc.load_gather(species_v, [j_vec])
            mi = si > 0
            mj = sj > 0
            s6 = jnp.where(mi, jnp.where(mj, s6_11, s6_10),
                           jnp.where(mj, s6_01, s6_00))
            fe = jnp.where(mi, jnp.where(mj, fe_11, fe_10),
                           jnp.where(mj, fe_01, fe_00))
            sh = jnp.where(mi, jnp.where(mj, sh_11, sh_10),
                           jnp.where(mj, sh_01, sh_00))
            r2 = x * x + y * y + z * z
            r6 = r2 * r2 * r2
            sr6 = s6 / r6
            e_half = fe * (sr6 * sr6 - sr6) - sh
            plsc.addupdate_scatter(energy_v, [i_vec], e_half)
            plsc.addupdate_scatter(energy_v, [j_vec], e_half)

    spec_copy.wait()
    for ci in range(N_CHUNKS):
        bi = ci % 2
        for d in descs.pop(ci):
            d.wait()
        process_chunk(bi)
        if ci + 2 < N_CHUNKS:
            descs[ci + 2] = start_chunk(ci + 2, bi)

    pltpu.sync_copy(energy_v, shared.at[s])
    plsc.subcore_barrier()
    pltpu.sync_copy(shared.at[:, pl.ds(s * SLICE, SLICE)], red_buf)

    @plsc.parallel_loop(0, SLICE // LANES, 1, unroll=4)
    def _(v):
        col = v * LANES
        acc = red_buf[0, pl.ds(col, LANES)]
        for r in range(1, NUM_SUBCORES):
            acc = acc + red_buf[r, pl.ds(col, LANES)]
        out_buf[pl.ds(col, LANES)] = acc

    pltpu.sync_copy(out_buf, out_hbm.at[c, pl.ds(s * SLICE, SLICE)])


@jax.jit
def _lj_sc(x, y, z, idx_i, idx_j, species, ptab):
    mesh = plsc.VectorSubcoreMesh(core_axis_name="c", subcore_axis_name="s")
    run = pl.kernel(
        _lj_kernel_body,
        out_type=jax.ShapeDtypeStruct((NUM_CORES, N_PAD), jnp.float32),
        mesh=mesh,
        compiler_params=pltpu.CompilerParams(needs_layout_passes=False),
        scratch_types=[
            pltpu.VMEM((N_NODES,), jnp.int32),
            pltpu.VMEM((N_PAD,), jnp.float32),
            pltpu.VMEM((24,), jnp.float32),
            pltpu.VMEM((CHUNK,), jnp.float32),
            pltpu.VMEM((CHUNK,), jnp.float32),
            pltpu.VMEM((CHUNK,), jnp.float32),
            pltpu.VMEM((CHUNK,), jnp.int32),
            pltpu.VMEM((CHUNK,), jnp.int32),
            pltpu.VMEM((CHUNK,), jnp.float32),
            pltpu.VMEM((CHUNK,), jnp.float32),
            pltpu.VMEM((CHUNK,), jnp.float32),
            pltpu.VMEM((CHUNK,), jnp.int32),
            pltpu.VMEM((CHUNK,), jnp.int32),
            pltpu.VMEM_SHARED((NUM_SUBCORES, N_PAD), jnp.float32),
            pltpu.VMEM((NUM_SUBCORES, SLICE), jnp.float32),
            pltpu.VMEM((SLICE,), jnp.float32),
            pltpu.SemaphoreType.DMA((2,)),
            pltpu.SemaphoreType.DMA,
        ],
    )
    return run(x, y, z, idx_i, idx_j, species, ptab)


def kernel(edge_vectors, edge_index, species, lj_table):
    sigma = lj_table[..., 0].reshape(-1)
    eps = lj_table[..., 1].reshape(-1)
    shift = lj_table[..., 2].reshape(-1)
    s3 = sigma * sigma * sigma
    ptab = jnp.zeros((24,), jnp.float32)
    ptab = ptab.at[0:4].set(s3 * s3)
    ptab = ptab.at[8:12].set(2.0 * eps)
    ptab = ptab.at[16:20].set(0.5 * shift)

    idx = edge_index.astype(jnp.int32)
    ev = edge_vectors.astype(jnp.float32)
    out = _lj_sc(ev[:, 0], ev[:, 1], ev[:, 2],
                 idx[0], idx[1],
                 species.astype(jnp.int32), ptab)
    energy = out[0, :N_NODES] + out[1, :N_NODES]
    return energy.reshape(-1, 1)
